# baseline (speedup 1.0000x reference)
import functools

import jax
import jax.numpy as jnp
from jax import lax
from jax.experimental import pallas as pl
from jax.experimental.pallas import tpu as pltpu
from jax.experimental.pallas import tpu_sc as plsc

BATCH = 16384
EMBED = 64
HIDDEN = 128
NC = 2
NS = 16
NW = NC * NS
B_PER_W = BATCH // NW
CHUNK = 128
N_CHUNK = B_PER_W // CHUNK
BLK = 2048


def _gather_body(lidx_hbm, cidx_hbm, ltab_hbm, ctab_hbm, el_out, ec_out,
                 idx_l, idx_c, rows_l, rows_c, sem):
    wid = lax.axis_index("s") * NC + lax.axis_index("c")
    base_row = wid * N_CHUNK
    pltpu.sync_copy(lidx_hbm.at[pl.ds(base_row, N_CHUNK)], idx_l)
    pltpu.sync_copy(cidx_hbm.at[pl.ds(base_row, N_CHUNK)], idx_c)
    copies = []
    for j in range(N_CHUNK):
        copies.append(pltpu.async_copy(
            ltab_hbm.at[idx_l.at[j]], rows_l.at[pl.ds(j * CHUNK, CHUNK)], sem))
        copies.append(pltpu.async_copy(
            ctab_hbm.at[idx_c.at[j]], rows_c.at[pl.ds(j * CHUNK, CHUNK)], sem))
    for c in copies:
        c.wait()
    base = wid * B_PER_W
    pltpu.sync_copy(rows_l, el_out.at[pl.ds(base, B_PER_W)])
    pltpu.sync_copy(rows_c, ec_out.at[pl.ds(base, B_PER_W)])


@functools.lru_cache(maxsize=None)
def _make_gather():
    return pl.kernel(
        _gather_body,
        mesh=plsc.VectorSubcoreMesh(core_axis_name="c", subcore_axis_name="s"),
        out_type=[
            jax.ShapeDtypeStruct((BATCH, EMBED), jnp.float32),
            jax.ShapeDtypeStruct((BATCH, EMBED), jnp.float32),
        ],
        scratch_types=[
            pltpu.VMEM((N_CHUNK, CHUNK), jnp.int32),
            pltpu.VMEM((N_CHUNK, CHUNK), jnp.int32),
            pltpu.VMEM((B_PER_W, EMBED), jnp.float32),
            pltpu.VMEM((B_PER_W, EMBED), jnp.float32),
            pltpu.SemaphoreType.DMA,
        ],
        compiler_params=pltpu.CompilerParams(use_tc_tiling_on_sc=False),
    )


def _mlp_body(el_ref, ec_ref, w1a_ref, w1b_ref, b1_ref, w2_ref, b2_ref,
              w3_ref, b3_ref, out_ref):
    h = jnp.dot(el_ref[...], w1a_ref[...], preferred_element_type=jnp.float32)
    h = h + jnp.dot(ec_ref[...], w1b_ref[...], preferred_element_type=jnp.float32)
    h = jnp.maximum(h + b1_ref[...], 0.0)
    h = jnp.maximum(
        jnp.dot(h, w2_ref[...], preferred_element_type=jnp.float32) + b2_ref[...],
        0.0)
    out_ref[...] = (
        jnp.dot(h, w3_ref[...], preferred_element_type=jnp.float32) + b3_ref[...])


def _mlp(el, ec, W1a, W1b, b1, W2, b2, W3, b3):
    return pl.pallas_call(
        _mlp_body,
        grid=(BATCH // BLK,),
        in_specs=[
            pl.BlockSpec((BLK, EMBED), lambda i: (i, 0)),
            pl.BlockSpec((BLK, EMBED), lambda i: (i, 0)),
            pl.BlockSpec((EMBED, HIDDEN), lambda i: (0, 0)),
            pl.BlockSpec((EMBED, HIDDEN), lambda i: (0, 0)),
            pl.BlockSpec((1, HIDDEN), lambda i: (0, 0)),
            pl.BlockSpec((HIDDEN, HIDDEN), lambda i: (0, 0)),
            pl.BlockSpec((1, HIDDEN), lambda i: (0, 0)),
            pl.BlockSpec((HIDDEN, 2), lambda i: (0, 0)),
            pl.BlockSpec((1, 2), lambda i: (0, 0)),
        ],
        out_specs=pl.BlockSpec((BLK, 2), lambda i: (i, 0)),
        out_shape=jax.ShapeDtypeStruct((BATCH, 2), jnp.float32),
    )(el, ec, W1a, W1b, b1, W2, b2, W3, b3)


def kernel(label_idx, category_idx, label_table, category_table,
           W1, b1, W2, b2, W3, b3):
    lidx = label_idx.astype(jnp.int32).reshape(BATCH // CHUNK, CHUNK)
    cidx = category_idx.astype(jnp.int32).reshape(BATCH // CHUNK, CHUNK)
    el, ec = _make_gather()(lidx, cidx, label_table, category_table)
    return _mlp(el, ec, W1[:EMBED], W1[EMBED:],
                b1.reshape(1, HIDDEN), W2, b2.reshape(1, HIDDEN),
                W3, b3.reshape(1, 2))


# speedup vs baseline: 1.4494x; 1.4494x over previous
import functools

import jax
import jax.numpy as jnp
from jax import lax
from jax.experimental import pallas as pl
from jax.experimental.pallas import tpu as pltpu
from jax.experimental.pallas import tpu_sc as plsc

BATCH = 16384
EMBED = 64
HIDDEN = 128
N_LAB = 1000000
N_CAT = 100000
NC = 2
NS = 16
NW = NC * NS
B_PER_W = BATCH // NW
CHUNK = 128
N_CHUNK = B_PER_W // CHUNK
BLK = 2048
PBLK = 4096


def _pack_body(tT_ref, out_ref):
    t = tT_ref[...].T
    out_ref[...] = jnp.concatenate([t, t], axis=1)


def _pack(tT, n_rows):
    grid = (n_rows + PBLK - 1) // PBLK
    return pl.pallas_call(
        _pack_body,
        grid=(grid,),
        in_specs=[pl.BlockSpec((EMBED, PBLK), lambda i: (0, i))],
        out_specs=pl.BlockSpec((PBLK, 2 * EMBED), lambda i: (i, 0)),
        out_shape=jax.ShapeDtypeStruct((n_rows, 2 * EMBED), jnp.float32),
    )(tT)


def _gather_body(lidx_hbm, cidx_hbm, ltab_hbm, ctab_hbm, el_out, ec_out,
                 idx_l, idx_c, rows, sem):
    wid = lax.axis_index("s") * NC + lax.axis_index("c")
    base_row = wid * N_CHUNK
    pltpu.sync_copy(lidx_hbm.at[pl.ds(base_row, N_CHUNK)], idx_l)
    pltpu.sync_copy(cidx_hbm.at[pl.ds(base_row, N_CHUNK)], idx_c)
    base = wid * B_PER_W
    copies = []
    for j in range(N_CHUNK):
        copies.append(pltpu.async_copy(
            ltab_hbm.at[idx_l.at[j]], rows.at[pl.ds(j * CHUNK, CHUNK)], sem))
    for c in copies:
        c.wait()
    pltpu.sync_copy(rows, el_out.at[pl.ds(base, B_PER_W)])
    copies = []
    for j in range(N_CHUNK):
        copies.append(pltpu.async_copy(
            ctab_hbm.at[idx_c.at[j]], rows.at[pl.ds(j * CHUNK, CHUNK)], sem))
    for c in copies:
        c.wait()
    pltpu.sync_copy(rows, ec_out.at[pl.ds(base, B_PER_W)])


@functools.lru_cache(maxsize=None)
def _make_gather():
    return pl.kernel(
        _gather_body,
        mesh=plsc.VectorSubcoreMesh(core_axis_name="c", subcore_axis_name="s"),
        out_type=[
            jax.ShapeDtypeStruct((BATCH, 2 * EMBED), jnp.float32),
            jax.ShapeDtypeStruct((BATCH, 2 * EMBED), jnp.float32),
        ],
        scratch_types=[
            pltpu.VMEM((N_CHUNK, CHUNK), jnp.int32),
            pltpu.VMEM((N_CHUNK, CHUNK), jnp.int32),
            pltpu.VMEM((B_PER_W, 2 * EMBED), jnp.float32),
            pltpu.SemaphoreType.DMA,
        ],
        compiler_params=pltpu.CompilerParams(use_tc_tiling_on_sc=True),
    )


def _mlp_body(el_ref, ec_ref, w1a_ref, w1b_ref, b1_ref, w2_ref, b2_ref,
              w3_ref, b3_ref, out_ref):
    h = jnp.dot(el_ref[:, :EMBED], w1a_ref[...],
                preferred_element_type=jnp.float32)
    h = h + jnp.dot(ec_ref[:, :EMBED], w1b_ref[...],
                    preferred_element_type=jnp.float32)
    h = jnp.maximum(h + b1_ref[...], 0.0)
    h = jnp.maximum(
        jnp.dot(h, w2_ref[...], preferred_element_type=jnp.float32) + b2_ref[...],
        0.0)
    out_ref[...] = (
        jnp.dot(h, w3_ref[...], preferred_element_type=jnp.float32) + b3_ref[...])


def _mlp(el, ec, W1a, W1b, b1, W2, b2, W3, b3):
    return pl.pallas_call(
        _mlp_body,
        grid=(BATCH // BLK,),
        in_specs=[
            pl.BlockSpec((BLK, 2 * EMBED), lambda i: (i, 0)),
            pl.BlockSpec((BLK, 2 * EMBED), lambda i: (i, 0)),
            pl.BlockSpec((EMBED, HIDDEN), lambda i: (0, 0)),
            pl.BlockSpec((EMBED, HIDDEN), lambda i: (0, 0)),
            pl.BlockSpec((1, HIDDEN), lambda i: (0, 0)),
            pl.BlockSpec((HIDDEN, HIDDEN), lambda i: (0, 0)),
            pl.BlockSpec((1, HIDDEN), lambda i: (0, 0)),
            pl.BlockSpec((HIDDEN, 2), lambda i: (0, 0)),
            pl.BlockSpec((1, 2), lambda i: (0, 0)),
        ],
        out_specs=pl.BlockSpec((BLK, 2), lambda i: (i, 0)),
        out_shape=jax.ShapeDtypeStruct((BATCH, 2), jnp.float32),
    )(el, ec, W1a, W1b, b1, W2, b2, W3, b3)


def kernel(label_idx, category_idx, label_table, category_table,
           W1, b1, W2, b2, W3, b3):
    lidx = label_idx.astype(jnp.int32).reshape(BATCH // CHUNK, CHUNK)
    cidx = category_idx.astype(jnp.int32).reshape(BATCH // CHUNK, CHUNK)
    ltp = _pack(label_table.T, N_LAB)
    ctp = _pack(category_table.T, N_CAT)
    el, ec = _make_gather()(lidx, cidx, ltp, ctp)
    return _mlp(el, ec, W1[:EMBED], W1[EMBED:],
                b1.reshape(1, HIDDEN), W2, b2.reshape(1, HIDDEN),
                W3, b3.reshape(1, 2))
